# baseline (speedup 1.0000x reference)
import functools

import jax
import jax.numpy as jnp
from jax import lax
from jax.experimental import pallas as pl
from jax.experimental.pallas import tpu as pltpu
from jax.experimental.pallas import tpu_sc as plsc

NUM_CORES = 2
NUM_SUBCORES = 16
NUM_WORKERS = NUM_CORES * NUM_SUBCORES
LANES = 16

BATCH = 16384
EMB_DIM = 64
ROWS_PER_WORKER = BATCH // NUM_WORKERS
BLOCKS_PER_WORKER = ROWS_PER_WORKER // LANES

_SC_PARAMS = pltpu.CompilerParams(
    needs_layout_passes=False, use_tc_tiling_on_sc=False)
_SC_MESH = plsc.VectorSubcoreMesh(
    core_axis_name="c", subcore_axis_name="s",
    num_cores=NUM_CORES, num_subcores=NUM_SUBCORES)


def _gather_u_body(uids_hbm, utab_hbm, ug_hbm, uidx_v, urows_v, usem):
    wid = lax.axis_index("s") * NUM_CORES + lax.axis_index("c")
    base = wid * ROWS_PER_WORKER
    pltpu.sync_copy(uids_hbm.at[pl.ds(base, ROWS_PER_WORKER)], uidx_v)
    pltpu.async_copy(utab_hbm.at[uidx_v], urows_v, usem).wait()
    pltpu.sync_copy(urows_v, ug_hbm.at[pl.ds(base, ROWS_PER_WORKER)])


_gather_u_kernel = functools.partial(
    pl.kernel,
    out_type=jax.ShapeDtypeStruct((BATCH, EMB_DIM), jnp.float32),
    mesh=_SC_MESH,
    compiler_params=_SC_PARAMS,
    scratch_types=[
        pltpu.VMEM((ROWS_PER_WORKER,), jnp.int32),
        pltpu.VMEM((ROWS_PER_WORKER, EMB_DIM), jnp.float32),
        pltpu.SemaphoreType.DMA,
    ],
)(_gather_u_body)


def _gmf_item_body(iids_hbm, itab_hbm, ug_hbm, wb_hbm, b_hbm, out_hbm,
                   iidx_v, urows_v, irows_v, b_v, wb_v, out_v, isem):
    wid = lax.axis_index("s") * NUM_CORES + lax.axis_index("c")
    base = wid * ROWS_PER_WORKER

    pltpu.sync_copy(iids_hbm.at[pl.ds(base, ROWS_PER_WORKER)], iidx_v)
    ci = pltpu.async_copy(itab_hbm.at[iidx_v], irows_v, isem)
    pltpu.sync_copy(ug_hbm.at[pl.ds(base, ROWS_PER_WORKER)], urows_v)
    pltpu.sync_copy(wb_hbm, wb_v)
    pltpu.sync_copy(b_hbm, b_v)
    ci.wait()

    bvec = b_v[:]
    iota = lax.iota(jnp.int32, LANES)

    def block(blk, carry):
        rows = blk * LANES + iota
        acc = jnp.zeros((LANES,), jnp.float32)
        for d in range(EMB_DIM):
            col = jnp.full((LANES,), d, jnp.int32)
            ucol = plsc.load_gather(urows_v, [rows, col])
            vcol = plsc.load_gather(irows_v, [rows, col])
            acc = acc + ucol * vcol * wb_v[pl.ds(d * LANES, LANES)]
        logits = acc + bvec
        out_v[pl.ds(blk * LANES, LANES)] = 1.0 / (1.0 + jnp.exp(-logits))
        return carry

    lax.fori_loop(0, BLOCKS_PER_WORKER, block, 0)

    pltpu.sync_copy(out_v, out_hbm.at[pl.ds(base, ROWS_PER_WORKER)])


_gmf_item_kernel = functools.partial(
    pl.kernel,
    out_type=jax.ShapeDtypeStruct((BATCH,), jnp.float32),
    mesh=_SC_MESH,
    compiler_params=_SC_PARAMS,
    scratch_types=[
        pltpu.VMEM((ROWS_PER_WORKER,), jnp.int32),
        pltpu.VMEM((ROWS_PER_WORKER, EMB_DIM), jnp.float32),
        pltpu.VMEM((ROWS_PER_WORKER, EMB_DIM), jnp.float32),
        pltpu.VMEM((LANES,), jnp.float32),
        pltpu.VMEM((EMB_DIM * LANES,), jnp.float32),
        pltpu.VMEM((ROWS_PER_WORKER,), jnp.float32),
        pltpu.SemaphoreType.DMA,
    ],
)(_gmf_item_body)


@jax.jit
def kernel(user_ids, item_ids, user_table, item_table, W, b):
    uids = user_ids.astype(jnp.int32)
    iids = item_ids.astype(jnp.int32)
    wb = jnp.broadcast_to(W.reshape(EMB_DIM, 1).astype(jnp.float32),
                          (EMB_DIM, LANES)).reshape(EMB_DIM * LANES)
    b16 = jnp.broadcast_to(b.astype(jnp.float32), (LANES,))
    ug = _gather_u_kernel(uids, user_table)
    out = _gmf_item_kernel(iids, item_table, ug, wb, b16)
    return out.reshape(BATCH, 1)


# speedup vs baseline: 1.5148x; 1.5148x over previous
import functools

import jax
import jax.numpy as jnp
from jax import lax
from jax.experimental import pallas as pl
from jax.experimental.pallas import tpu as pltpu
from jax.experimental.pallas import tpu_sc as plsc

NUM_CORES = 2
NUM_SUBCORES = 16
NUM_WORKERS = NUM_CORES * NUM_SUBCORES
LANES = 16

BATCH = 16384
EMB_DIM = 64
NUM_FULL_TILES = 7812
TAIL_BASE = NUM_FULL_TILES * 128
CAPL = 8
ID_CHUNK = 1024
ROWS_PER_WORKER = BATCH // NUM_WORKERS
CHUNK = 128
NUM_CHUNKS = ROWS_PER_WORKER // CHUNK
BLOCKS_PER_CHUNK = CHUNK // LANES

_SC_PARAMS = pltpu.CompilerParams(
    needs_layout_passes=False, use_tc_tiling_on_sc=True)
_SC_MESH = plsc.VectorSubcoreMesh(
    core_axis_name="c", subcore_axis_name="s",
    num_cores=NUM_CORES, num_subcores=NUM_SUBCORES)

_NT_BIG = 246
_NT_SMALL = 244
_MAX_NT = _NT_BIG
_BUCKET_TILES = _NT_BIG + 1


def _gather_body(uids_hbm, iids_hbm, ut_hbm, it_hbm, tailu_hbm, tailv_hbm,
                 ug_hbm, vg_hbm,
                 uids_v, iids_v, cu_v, ci_v, bu_v, bi_v,
                 ubufA, ubufB, vbufA, vbufB, stage_v,
                 semUA, semUB, semVA, semVB, semT,
                 s0, s1, s2, s3, s4, s5, s6, s7,
                 s8, s9, s10, s11, s12, s13, s14, s15):
    lane_sems = [s0, s1, s2, s3, s4, s5, s6, s7,
                 s8, s9, s10, s11, s12, s13, s14, s15]
    wid = lax.axis_index("s") * NUM_CORES + lax.axis_index("c")
    is_big = wid < 2
    t0 = jnp.where(is_big, _NT_BIG * wid, 2 * _NT_BIG + _NT_SMALL * (wid - 2))
    nt = jnp.where(is_big, _NT_BIG, _NT_SMALL)
    is_last = (wid == NUM_WORKERS - 1).astype(jnp.int32)

    iota = lax.iota(jnp.int32, LANES)
    zeros16 = jnp.zeros((LANES,), jnp.int32)

    def zero_counts(i, c):
        cu_v[pl.ds(i * LANES, LANES)] = zeros16
        ci_v[pl.ds(i * LANES, LANES)] = zeros16
        return c

    lax.fori_loop(0, _BUCKET_TILES, zero_counts, 0)

    def bucket_one(ids_v, counts_v, buckets_v, j, kbase):
        idv = ids_v[pl.ds(j * LANES, LANES)]
        t = lax.shift_right_logical(idv, 7)
        mine = (t >= t0) & (t < t0 + nt + is_last)
        tloc = jnp.where(mine, t - t0, 0)
        k16 = kbase + j * LANES + iota
        pay = lax.shift_left(k16, 7) | (idv & 127)
        cidx = tloc * LANES + iota
        cnt = plsc.load_gather(counts_v, [cidx], mask=mine)
        cnt = jnp.where(mine, cnt, CAPL)
        ok = mine & (cnt < CAPL)
        slotaddr = cidx * CAPL + jnp.where(ok, cnt, 0)
        plsc.store_scatter(buckets_v, [slotaddr], pay, mask=ok)
        plsc.addupdate_scatter(counts_v, [cidx], jnp.ones((LANES,), jnp.int32),
                               mask=ok)

    def bucket_chunk(ci, c):
        kbase = ci * ID_CHUNK
        pltpu.sync_copy(uids_hbm.at[pl.ds(kbase, ID_CHUNK)], uids_v)
        pltpu.sync_copy(iids_hbm.at[pl.ds(kbase, ID_CHUNK)], iids_v)

        def bucket_step(j, c2):
            bucket_one(uids_v, cu_v, bu_v, j, kbase)
            bucket_one(iids_v, ci_v, bi_v, j, kbase)
            return c2

        lax.fori_loop(0, ID_CHUNK // LANES, bucket_step, 0)
        return c

    lax.fori_loop(0, BATCH // ID_CHUNK, bucket_chunk, 0)

    for L in range(LANES):
        pltpu.async_copy(ug_hbm.at[0], stage_v.at[L], lane_sems[L])

    def fire(tile, ubuf, vbuf, semU, semV):
        off = pl.multiple_of(tile * 128, 128)
        pltpu.async_copy(ut_hbm.at[:, pl.ds(off, 128)], ubuf, semU)
        pltpu.async_copy(it_hbm.at[:, pl.ds(off, 128)], vbuf, semV)

    def wait_buf(ubuf, vbuf, semU, semV):
        pltpu.make_async_copy(
            ut_hbm.at[:, pl.ds(0, 128)], ubuf, semU).wait()
        pltpu.make_async_copy(
            it_hbm.at[:, pl.ds(0, 128)], vbuf, semV).wait()

    def extract_tile(tloc, ubuf, vbuf):
        for (win, counts_v, buckets_v, out_hbm) in (
                (ubuf, cu_v, bu_v, ug_hbm),
                (vbuf, ci_v, bi_v, vg_hbm)):
            cl16 = counts_v[pl.ds(tloc * LANES, LANES)]
            for L in range(LANES):
                cnt = cl16[L]
                base = (tloc * LANES + L) * CAPL

                def body(e, c, win=win, out_hbm=out_hbm, base=base, L=L):
                    pltpu.make_async_copy(
                        ug_hbm.at[0], stage_v.at[L], lane_sems[L]).wait()
                    ent = buckets_v[pl.ds(base + e, 16)][0]
                    k = lax.shift_right_logical(ent, 7)
                    lane = ent & 127
                    lanev = jnp.full((LANES,), 0, jnp.int32) + lane
                    for q in range(4):
                        colq = plsc.load_gather(
                            win, [iota + q * LANES, lanev])
                        stage_v[L, pl.ds(q * LANES, LANES)] = colq
                    pltpu.async_copy(
                        stage_v.at[L], out_hbm.at[k], lane_sems[L])
                    return c

                lax.fori_loop(0, cnt, body, 0)

    fire(t0, ubufA, vbufA, semUA, semVA)

    def pair_step(p, c):
        g0 = t0 + 2 * p
        fire(g0 + 1, ubufB, vbufB, semUB, semVB)
        wait_buf(ubufA, vbufA, semUA, semVA)
        extract_tile(g0 - t0, ubufA, vbufA)
        nxt = jnp.where(g0 + 2 < t0 + nt, g0 + 2, t0)
        fire(nxt, ubufA, vbufA, semUA, semVA)
        wait_buf(ubufB, vbufB, semUB, semVB)
        extract_tile(g0 + 1 - t0, ubufB, vbufB)
        return c

    lax.fori_loop(0, nt // 2, pair_step, 0)
    wait_buf(ubufA, vbufA, semUA, semVA)

    for (tail_hbm, counts_v, buckets_v, out_hbm) in (
            (tailu_hbm, cu_v, bu_v, ug_hbm),
            (tailv_hbm, ci_v, bi_v, vg_hbm)):
        cl16 = counts_v[pl.ds(nt * LANES, LANES)]
        for L in range(LANES):
            cnt = cl16[L]
            base = (nt * LANES + L) * CAPL

            def tbody(e, c, tail_hbm=tail_hbm, out_hbm=out_hbm,
                      base=base, L=L):
                pltpu.make_async_copy(
                    ug_hbm.at[0], stage_v.at[L], lane_sems[L]).wait()
                ent = buckets_v[pl.ds(base + e, 16)][0]
                k = lax.shift_right_logical(ent, 7)
                row = ent & 127
                pltpu.async_copy(
                    tail_hbm.at[row], stage_v.at[L], semT).wait()
                pltpu.async_copy(
                    stage_v.at[L], out_hbm.at[k], lane_sems[L])
                return c

            lax.fori_loop(0, cnt, tbody, 0)

    for L in range(LANES):
        pltpu.make_async_copy(
            ug_hbm.at[0], stage_v.at[L], lane_sems[L]).wait()


_gather_kernel = functools.partial(
    pl.kernel,
    out_type=(jax.ShapeDtypeStruct((BATCH, EMB_DIM), jnp.float32),
              jax.ShapeDtypeStruct((BATCH, EMB_DIM), jnp.float32)),
    mesh=_SC_MESH,
    compiler_params=_SC_PARAMS,
    scratch_types=[
        pltpu.VMEM((ID_CHUNK,), jnp.int32),
        pltpu.VMEM((ID_CHUNK,), jnp.int32),
        pltpu.VMEM((_BUCKET_TILES * LANES + 16,), jnp.int32),
        pltpu.VMEM((_BUCKET_TILES * LANES + 16,), jnp.int32),
        pltpu.VMEM((_BUCKET_TILES * LANES * CAPL + 16,), jnp.int32),
        pltpu.VMEM((_BUCKET_TILES * LANES * CAPL + 16,), jnp.int32),
        pltpu.VMEM((EMB_DIM, 128), jnp.float32),
        pltpu.VMEM((EMB_DIM, 128), jnp.float32),
        pltpu.VMEM((EMB_DIM, 128), jnp.float32),
        pltpu.VMEM((EMB_DIM, 128), jnp.float32),
        pltpu.VMEM((LANES, EMB_DIM), jnp.float32),
        pltpu.SemaphoreType.DMA,
        pltpu.SemaphoreType.DMA,
        pltpu.SemaphoreType.DMA,
        pltpu.SemaphoreType.DMA,
        pltpu.SemaphoreType.DMA,
    ] + [pltpu.SemaphoreType.DMA] * LANES,
)(_gather_body)


def _compute_body(ug_hbm, vg_hbm, wb_hbm, b_hbm, out_hbm,
                  urows_v, irows_v, b_v, wb_v, out_v):
    wid = lax.axis_index("s") * NUM_CORES + lax.axis_index("c")
    base = wid * ROWS_PER_WORKER

    pltpu.sync_copy(wb_hbm, wb_v)
    pltpu.sync_copy(b_hbm, b_v)
    bvec = b_v[:]
    iota = lax.iota(jnp.int32, LANES)

    def block(blk, carry):
        rows = (blk % BLOCKS_PER_CHUNK) * LANES + iota
        acc = jnp.zeros((LANES,), jnp.float32)
        for d in range(EMB_DIM):
            col = jnp.full((LANES,), d, jnp.int32)
            ucol = plsc.load_gather(urows_v, [rows, col])
            vcol = plsc.load_gather(irows_v, [rows, col])
            acc = acc + ucol * vcol * wb_v[pl.ds(d * LANES, LANES)]
        logits = acc + bvec
        out_v[pl.ds(blk * LANES, LANES)] = 1.0 / (1.0 + jnp.exp(-logits))
        return carry

    for ch in range(NUM_CHUNKS):
        pltpu.sync_copy(
            ug_hbm.at[pl.ds(base + ch * CHUNK, CHUNK)], urows_v)
        pltpu.sync_copy(
            vg_hbm.at[pl.ds(base + ch * CHUNK, CHUNK)], irows_v)
        lax.fori_loop(ch * BLOCKS_PER_CHUNK, (ch + 1) * BLOCKS_PER_CHUNK,
                      block, 0)

    pltpu.sync_copy(out_v, out_hbm.at[pl.ds(base, ROWS_PER_WORKER)])


_compute_kernel = functools.partial(
    pl.kernel,
    out_type=jax.ShapeDtypeStruct((BATCH,), jnp.float32),
    mesh=_SC_MESH,
    compiler_params=_SC_PARAMS,
    scratch_types=[
        pltpu.VMEM((CHUNK, EMB_DIM), jnp.float32),
        pltpu.VMEM((CHUNK, EMB_DIM), jnp.float32),
        pltpu.VMEM((LANES,), jnp.float32),
        pltpu.VMEM((EMB_DIM * LANES,), jnp.float32),
        pltpu.VMEM((ROWS_PER_WORKER,), jnp.float32),
    ],
)(_compute_body)


@jax.jit
def kernel(user_ids, item_ids, user_table, item_table, W, b):
    uids = user_ids.astype(jnp.int32)
    iids = item_ids.astype(jnp.int32)
    ut_t = user_table.T
    it_t = item_table.T
    tail_u = lax.slice(user_table, (TAIL_BASE, 0), (1000000, EMB_DIM))
    tail_v = lax.slice(item_table, (TAIL_BASE, 0), (1000000, EMB_DIM))
    wb = jnp.broadcast_to(W.reshape(EMB_DIM, 1).astype(jnp.float32),
                          (EMB_DIM, LANES)).reshape(EMB_DIM * LANES)
    b16 = jnp.broadcast_to(b.astype(jnp.float32), (LANES,))
    ug, vg = _gather_kernel(uids, iids, ut_t, it_t, tail_u, tail_v)
    out = _compute_kernel(ug, vg, wb, b16)
    return out.reshape(BATCH, 1)
